# baseline (speedup 1.0000x reference)
import functools

import jax
import jax.numpy as jnp
from jax import lax
from jax.experimental import pallas as pl
from jax.experimental.pallas import tpu as pltpu
from jax.experimental.pallas import tpu_sc as plsc

_B = 4096
_L = 200
_H = 32
_N = _B * _L
_NC = 2
_NS = 16
_NW = _NC * _NS
_NT = _B // 128
_NU = _L * _NT
_UPW = _NU // _NW


def _tok_pos_body(xT_hbm, emb_hbm, pos_hbm, out_hbm,
                  idxA, idxB, rowsA, rowsB, blkA, blkB, pos_v,
                  sgA, sgB, soA, soB):
    wid = lax.axis_index("s") * _NC + lax.axis_index("c")
    u0 = wid * _UPW
    pltpu.sync_copy(pos_hbm, pos_v)

    iota = lax.iota(jnp.int32, 16)
    z16 = jnp.zeros((16,), jnp.int32)
    g_lo = lax.shift_right_logical(iota, 3)
    g_hi = lax.shift_right_logical(iota + 16, 3)
    in_lo = lax.bitwise_and(iota, 7) * 128
    in_hi = in_lo

    def unit_lt(u):
        return lax.shift_right_logical(u, 5), lax.bitwise_and(u, 31)

    def load_idx(u, idxbuf):
        l, t = unit_lt(u)
        off = pl.multiple_of(l * _B + t * 128, 128)
        pltpu.sync_copy(xT_hbm.at[pl.ds(off, 128)], idxbuf)

    def gather_start(idxbuf, rowsbuf, sem):
        pltpu.async_copy(emb_hbm.at[idxbuf], rowsbuf, sem)

    def gather_wait(idxbuf, rowsbuf, sem):
        pltpu.make_async_copy(emb_hbm.at[idxbuf], rowsbuf, sem).wait()

    def out_start(u, blk, sem):
        l, t = unit_lt(u)
        pltpu.async_copy(blk, out_hbm.at[pl.ds(l * 4, 4), pl.ds(t, 1), :], sem)

    def out_wait(blk, sem):
        pltpu.make_async_copy(
            blk, out_hbm.at[pl.ds(0, 4), pl.ds(0, 1), :], sem).wait()

    def compute(u, rowsbuf, blk):
        l, _ = unit_lt(u)
        p_lo = pos_v[l, pl.ds(0, 16)]
        p_hi = pos_v[l, pl.ds(16, 16)]

        def cbody(c, carry):
            cs = jnp.full((16,), c, jnp.int32)
            plsc.store_scatter(blk, [g_lo, z16, in_lo + cs],
                               rowsbuf[c, pl.ds(0, 16)] + p_lo)
            plsc.store_scatter(blk, [g_hi, z16, in_hi + cs],
                               rowsbuf[c, pl.ds(16, 16)] + p_hi)
            return carry

        lax.fori_loop(0, 128, cbody, 0, unroll=4)

    load_idx(u0, idxA)
    gather_start(idxA, rowsA, sgA)

    def pair_body(g2, carry):
        u = u0 + 2 * g2
        gather_wait(idxA, rowsA, sgA)
        load_idx(u + 1, idxB)
        gather_start(idxB, rowsB, sgB)

        @pl.when(g2 > 0)
        def _():
            out_wait(blkA, soA)

        compute(u, rowsA, blkA)
        out_start(u, blkA, soA)

        @pl.when(g2 < _UPW // 2 - 1)
        def _():
            load_idx(u + 2, idxA)
            gather_start(idxA, rowsA, sgA)

        gather_wait(idxB, rowsB, sgB)

        @pl.when(g2 > 0)
        def _():
            out_wait(blkB, soB)

        compute(u + 1, rowsB, blkB)
        out_start(u + 1, blkB, soB)
        return carry

    lax.fori_loop(0, _UPW // 2, pair_body, 0)
    out_wait(blkA, soA)
    out_wait(blkB, soB)


def kernel(x, emb_table, pos_table):
    xT_flat = x.T.reshape(_N)
    mesh = plsc.VectorSubcoreMesh(core_axis_name="c", subcore_axis_name="s")
    call = functools.partial(
        pl.kernel,
        mesh=mesh,
        compiler_params=pltpu.CompilerParams(
            use_tc_tiling_on_sc=False, needs_layout_passes=False),
        out_type=jax.ShapeDtypeStruct((_L * 4, _NT, 1024), jnp.float32),
        scratch_types=[
            pltpu.VMEM((128,), jnp.int32),
            pltpu.VMEM((128,), jnp.int32),
            pltpu.VMEM((128, _H), jnp.float32),
            pltpu.VMEM((128, _H), jnp.float32),
            pltpu.VMEM((4, 1, 1024), jnp.float32),
            pltpu.VMEM((4, 1, 1024), jnp.float32),
            pltpu.VMEM((_L, _H), jnp.float32),
            pltpu.SemaphoreType.DMA,
            pltpu.SemaphoreType.DMA,
            pltpu.SemaphoreType.DMA,
            pltpu.SemaphoreType.DMA,
        ],
    )(_tok_pos_body)
    out = call(xT_flat, emb_table, pos_table)
    v = out.reshape(_L, 4, _NT, 8, 128)
    return v.transpose(2, 4, 0, 1, 3).reshape(_B, _L, _H)


# speedup vs baseline: 1.0104x; 1.0104x over previous
import functools

import jax
import jax.numpy as jnp
from jax import lax
from jax.experimental import pallas as pl
from jax.experimental.pallas import tpu as pltpu
from jax.experimental.pallas import tpu_sc as plsc

_B = 4096
_L = 200
_H = 32
_NC = 2
_NS = 16
_NW = _NC * _NS


def _tok_pos_body(xT_hbm, emb_hbm, pos_hbm, out_hbm,
                  idx_all, rowsA, rowsB, blkA, blkB, pos_v,
                  sgA, sgB, soA, soB):
    wid = lax.axis_index("s") * _NC + lax.axis_index("c")
    pltpu.sync_copy(pos_hbm, pos_v)
    pltpu.sync_copy(xT_hbm.at[:, pl.ds(wid * 128, 128)], idx_all)

    iota = lax.iota(jnp.int32, 16)
    c_vecs = [iota + 16 * k for k in range(8)]

    def gather_start(l, rowsbuf, sem):
        pltpu.async_copy(emb_hbm.at[idx_all.at[l]], rowsbuf, sem)

    def gather_wait(rowsbuf, sem):
        pltpu.make_async_copy(emb_hbm.at[idx_all.at[0]], rowsbuf, sem).wait()

    def out_start(l, blk, sem):
        pltpu.async_copy(
            blk, out_hbm.at[pl.ds(l * 4, 4), pl.ds(wid, 1), :], sem)

    def out_wait(blk, sem):
        pltpu.make_async_copy(
            blk, out_hbm.at[pl.ds(0, 4), pl.ds(0, 1), :], sem).wait()

    def compute(l, rowsbuf, blk):
        p_lo = pos_v[l, pl.ds(0, 16)]
        p_hi = pos_v[l, pl.ds(16, 16)]

        def cbody(c, carry):
            rowsbuf[c, pl.ds(0, 16)] = rowsbuf[c, pl.ds(0, 16)] + p_lo
            rowsbuf[c, pl.ds(16, 16)] = rowsbuf[c, pl.ds(16, 16)] + p_hi
            return carry

        lax.fori_loop(0, 128, cbody, 0, unroll=8)

        def hbody(h, carry):
            hs = jnp.full((16,), h, jnp.int32)
            g = lax.shift_right_logical(h, 3)
            base = lax.bitwise_and(h, 7) * 128
            for k in range(8):
                blk[g, 0, pl.ds(base + 16 * k, 16)] = plsc.load_gather(
                    rowsbuf, [c_vecs[k], hs])
            return carry

        lax.fori_loop(0, _H, hbody, 0, unroll=2)

    gather_start(0, rowsA, sgA)

    def pair_body(l2, carry):
        l = 2 * l2
        gather_wait(rowsA, sgA)
        gather_start(l + 1, rowsB, sgB)

        @pl.when(l2 > 0)
        def _():
            out_wait(blkA, soA)

        compute(l, rowsA, blkA)
        out_start(l, blkA, soA)

        @pl.when(l2 < _L // 2 - 1)
        def _():
            gather_start(l + 2, rowsA, sgA)

        gather_wait(rowsB, sgB)

        @pl.when(l2 > 0)
        def _():
            out_wait(blkB, soB)

        compute(l + 1, rowsB, blkB)
        out_start(l + 1, blkB, soB)
        return carry

    lax.fori_loop(0, _L // 2, pair_body, 0)
    out_wait(blkA, soA)
    out_wait(blkB, soB)


def kernel(x, emb_table, pos_table):
    xT = x.T
    mesh = plsc.VectorSubcoreMesh(core_axis_name="c", subcore_axis_name="s")
    call = functools.partial(
        pl.kernel,
        mesh=mesh,
        compiler_params=pltpu.CompilerParams(
            use_tc_tiling_on_sc=False, needs_layout_passes=False),
        out_type=jax.ShapeDtypeStruct((_L * 4, _NW, 1024), jnp.float32),
        scratch_types=[
            pltpu.VMEM((_L, 128), jnp.int32),
            pltpu.VMEM((128, _H), jnp.float32),
            pltpu.VMEM((128, _H), jnp.float32),
            pltpu.VMEM((4, 1, 1024), jnp.float32),
            pltpu.VMEM((4, 1, 1024), jnp.float32),
            pltpu.VMEM((_L, _H), jnp.float32),
            pltpu.SemaphoreType.DMA,
            pltpu.SemaphoreType.DMA,
            pltpu.SemaphoreType.DMA,
            pltpu.SemaphoreType.DMA,
        ],
    )(_tok_pos_body)
    out = call(xT, emb_table, pos_table)
    v = out.reshape(_L, 4, _NW, 8, 128)
    return v.transpose(2, 4, 0, 1, 3).reshape(_B, _L, _H)


# speedup vs baseline: 1.3494x; 1.3355x over previous
import functools

import jax
import jax.numpy as jnp
from jax import lax
from jax.experimental import pallas as pl
from jax.experimental.pallas import tpu as pltpu
from jax.experimental.pallas import tpu_sc as plsc

_B = 4096
_L = 200
_H = 32
_NC = 2
_NS = 16
_NW = _NC * _NS


def _tok_pos_body(xT_hbm, emb_hbm, pos_hbm, out_hbm,
                  idx_all, rowsA, rowsB, blkA, blkB, pos_v,
                  sgA, sgB, soA, soB):
    wid = lax.axis_index("s") * _NC + lax.axis_index("c")
    pltpu.sync_copy(pos_hbm, pos_v)
    pltpu.sync_copy(xT_hbm.at[:, pl.ds(wid * 128, 128)], idx_all)

    iota = lax.iota(jnp.int32, 16)
    c_vecs = [iota + 16 * k for k in range(8)]

    def gather_start(l, rowsbuf, sem):
        pltpu.async_copy(emb_hbm.at[idx_all.at[l]], rowsbuf, sem)

    def gather_wait(rowsbuf, sem):
        pltpu.make_async_copy(emb_hbm.at[idx_all.at[0]], rowsbuf, sem).wait()

    def out_start(l, blk, sem):
        pltpu.async_copy(
            blk, out_hbm.at[pl.ds(l * 4, 4), pl.ds(wid, 1), :], sem)

    def out_wait(blk, sem):
        pltpu.make_async_copy(
            blk, out_hbm.at[pl.ds(0, 4), pl.ds(0, 1), :], sem).wait()

    def compute(l, rowsbuf, blk):
        p_lo = pos_v[l, pl.ds(0, 16)]
        p_hi = pos_v[l, pl.ds(16, 16)]

        @plsc.parallel_loop(0, 128, unroll=8)
        def _(c):
            rowsbuf[c, pl.ds(0, 16)] = rowsbuf[c, pl.ds(0, 16)] + p_lo
            rowsbuf[c, pl.ds(16, 16)] = rowsbuf[c, pl.ds(16, 16)] + p_hi

        @plsc.parallel_loop(0, _H, unroll=4)
        def _(h):
            hs = jnp.full((16,), h, jnp.int32)
            g = lax.shift_right_logical(h, 3)
            base = lax.bitwise_and(h, 7) * 128
            for k in range(8):
                blk[g, 0, pl.ds(base + 16 * k, 16)] = plsc.load_gather(
                    rowsbuf, [c_vecs[k], hs])

    gather_start(0, rowsA, sgA)

    def pair_body(l2, carry):
        l = 2 * l2
        gather_wait(rowsA, sgA)
        gather_start(l + 1, rowsB, sgB)

        @pl.when(l2 > 0)
        def _():
            out_wait(blkA, soA)

        compute(l, rowsA, blkA)
        out_start(l, blkA, soA)

        @pl.when(l2 < _L // 2 - 1)
        def _():
            gather_start(l + 2, rowsA, sgA)

        gather_wait(rowsB, sgB)

        @pl.when(l2 > 0)
        def _():
            out_wait(blkB, soB)

        compute(l + 1, rowsB, blkB)
        out_start(l + 1, blkB, soB)
        return carry

    lax.fori_loop(0, _L // 2, pair_body, 0)
    out_wait(blkA, soA)
    out_wait(blkB, soB)


def kernel(x, emb_table, pos_table):
    xT = x.T
    mesh = plsc.VectorSubcoreMesh(core_axis_name="c", subcore_axis_name="s")
    call = functools.partial(
        pl.kernel,
        mesh=mesh,
        compiler_params=pltpu.CompilerParams(
            use_tc_tiling_on_sc=False, needs_layout_passes=False),
        out_type=jax.ShapeDtypeStruct((_L * 4, _NW, 1024), jnp.float32),
        scratch_types=[
            pltpu.VMEM((_L, 128), jnp.int32),
            pltpu.VMEM((128, _H), jnp.float32),
            pltpu.VMEM((128, _H), jnp.float32),
            pltpu.VMEM((4, 1, 1024), jnp.float32),
            pltpu.VMEM((4, 1, 1024), jnp.float32),
            pltpu.VMEM((_L, _H), jnp.float32),
            pltpu.SemaphoreType.DMA,
            pltpu.SemaphoreType.DMA,
            pltpu.SemaphoreType.DMA,
            pltpu.SemaphoreType.DMA,
        ],
    )(_tok_pos_body)
    out = call(xT, emb_table, pos_table)
    v = out.reshape(_L, 4, _NW, 8, 128)
    return v.transpose(2, 4, 0, 1, 3).reshape(_B, _L, _H)
